# baseline (speedup 1.0000x reference)
import jax
import jax.numpy as jnp
from jax.experimental import pallas as pl
from jax.experimental.pallas import tpu as pltpu


_GB = 8
_SB = 64


def _l2norm_rows0(x):
    s = jnp.sum(x * x, axis=0, keepdims=True)
    return x * jax.lax.rsqrt(jnp.maximum(s, 1e-12))


def _tnull_body(w_ref, worlds_ref, t_ref):
    w = w_ref[...]
    x = jax.lax.dot_general(w, worlds_ref[...], (((2,), (1,)), ((), ())),
                            preferred_element_type=jnp.float32)
    s = jnp.sum(x * x, axis=1, keepdims=True)
    t_ref[...] = x * jax.lax.rsqrt(jnp.maximum(s, 1e-12))


def _make_main_body(P, B, D, NW, Gb):
    def body(startsN_ref, startsU_ref, startsB_ref,
             symN_ref, rowN_ref,
             symU_ref, a0U_ref, rowU_ref,
             symB_ref, a0B_ref, a1B_ref, rowB_ref,
             cs_ref, tn_ref, wun_ref, bun_ref, wbin_ref, bbin_ref,
             out_ref):
        j = pl.program_id(0)
        base = j * Gb
        out_ref[...] = jnp.zeros((Gb, D, NW), jnp.float32)

        def clb(lb):
            return jnp.minimum(jnp.maximum(lb, 0), Gb - 1)

        def pair_loop(lo, hi, nmax, compute_y, row_ref):
            def one(i, valid):
                y = compute_y(i)
                lb = clb(row_ref[i] - base)

                @pl.when(valid)
                def _():
                    out_ref[pl.ds(lb, 1)] = out_ref[pl.ds(lb, 1)] + y[None]

            def pbody(k, carry):
                i = lo + 2 * k
                i2 = jnp.minimum(i + 1, nmax - 1)
                one(i, i < hi)
                one(i2, i + 1 < hi)
                return carry

            jax.lax.fori_loop(0, (hi - lo + 1) // 2, pbody, 0, unroll=False)

        def null_y(i):
            s = symN_ref[i]
            return tn_ref[pl.ds(s, 1)][0]

        def unary_y(i):
            s = symU_ref[i]
            a = a0U_ref[i]
            lb = clb(rowU_ref[i] - base)
            x = cs_ref[pl.ds(a, 1), pl.ds(lb, 1)][0, 0]
            w = wun_ref[pl.ds(s, 1)][0]
            y = jax.lax.dot_general(w, x, (((1,), (0,)), ((), ())),
                                    preferred_element_type=jnp.float32)
            b = bun_ref[pl.ds(s, 1)][0]
            y = y + jax.lax.broadcast_in_dim(b, (D, NW), (0,))
            return _l2norm_rows0(y)

        def binary_y(i):
            s = symB_ref[i]
            a0 = a0B_ref[i]
            a1 = a1B_ref[i]
            lb = clb(rowB_ref[i] - base)
            xl = cs_ref[pl.ds(a0, 1), pl.ds(lb, 1)][0, 0]
            xr = cs_ref[pl.ds(a1, 1), pl.ds(lb, 1)][0, 0]
            w = wbin_ref[pl.ds(s, 1)][0]
            y = (jax.lax.dot_general(w[:, :D], xl, (((1,), (0,)), ((), ())),
                                     preferred_element_type=jnp.float32)
                 + jax.lax.dot_general(w[:, D:], xr, (((1,), (0,)), ((), ())),
                                       preferred_element_type=jnp.float32))
            b = bbin_ref[pl.ds(s, 1)][0]
            y = y + jax.lax.broadcast_in_dim(b, (D, NW), (0,))
            return _l2norm_rows0(y)

        n0 = symN_ref.shape[0]
        n1 = symU_ref.shape[0]
        n2 = symB_ref.shape[0]
        pair_loop(startsN_ref[j], startsN_ref[j + 1], n0, null_y, rowN_ref)
        pair_loop(startsU_ref[j], startsU_ref[j + 1], n1, unary_y, rowU_ref)
        pair_loop(startsB_ref[j], startsB_ref[j + 1], n2, binary_y, rowB_ref)

    return body


def kernel(worlds, computed_states, null_indices, null_symbols,
           unary_indices, unary_symbols, unary_args,
           binary_indices, binary_symbols, binary_args,
           W_null, W_un, b_un, W_bin, b_bin):
    P, B, D, NW = computed_states.shape
    S = W_null.shape[0]
    i32 = jnp.int32
    Gb = _GB
    nblk = B // Gb

    T_null = pl.pallas_call(
        _tnull_body,
        grid=(S // _SB,),
        in_specs=[
            pl.BlockSpec((_SB, D, D), lambda i: (i, 0, 0)),
            pl.BlockSpec((NW, D), lambda i: (0, 0)),
        ],
        out_specs=pl.BlockSpec((_SB, D, NW), lambda i: (i, 0, 0)),
        out_shape=jax.ShapeDtypeStruct((S, D, NW), jnp.float32),
        name="sat3_tnull",
    )(W_null, worlds)

    def prep_packed(idx, payloads, widths):
        key = idx.astype(i32)
        for p, w in zip(payloads, widths):
            key = (key << w) | p.astype(i32)
        key = jnp.sort(key)
        tot = sum(widths)
        row = key >> tot
        starts = jnp.searchsorted(
            row, jnp.arange(0, B + 1, Gb, dtype=i32)).astype(i32)
        outs = []
        rem = key
        for w in reversed(widths):
            outs.append(rem & ((1 << w) - 1))
            rem = rem >> w
        return (starts, row) + tuple(reversed(outs))

    sym_bits = max(1, (S - 1).bit_length())
    arg_bits = max(1, (P - 1).bit_length())
    startsN, rowN, symN = prep_packed(
        null_indices, [null_symbols], [sym_bits])
    startsU, rowU, symU, a0U = prep_packed(
        unary_indices, [unary_symbols, unary_args], [sym_bits, arg_bits])
    startsB, rowB, symB, a0B, a1B = prep_packed(
        binary_indices, [binary_symbols, binary_args[:, 0], binary_args[:, 1]],
        [sym_bits, arg_bits, arg_bits])

    grid_spec = pltpu.PrefetchScalarGridSpec(
        num_scalar_prefetch=12,
        grid=(nblk,),
        in_specs=[
            pl.BlockSpec((P, Gb, D, NW),
                         lambda j, *_: (0, j, 0, 0)),
            pl.BlockSpec(memory_space=pltpu.MemorySpace.VMEM),
            pl.BlockSpec(memory_space=pltpu.MemorySpace.VMEM),
            pl.BlockSpec(memory_space=pltpu.MemorySpace.VMEM),
            pl.BlockSpec(memory_space=pltpu.MemorySpace.VMEM),
            pl.BlockSpec(memory_space=pltpu.MemorySpace.VMEM),
        ],
        out_specs=pl.BlockSpec((Gb, D, NW), lambda j, *_: (j, 0, 0)),
    )

    out = pl.pallas_call(
        _make_main_body(P, B, D, NW, Gb),
        grid_spec=grid_spec,
        out_shape=jax.ShapeDtypeStruct((B, D, NW), jnp.float32),
        name="sat3_main",
    )(startsN, startsU, startsB, symN, rowN, symU, a0U, rowU,
      symB, a0B, a1B, rowB,
      computed_states, T_null, W_un, b_un, W_bin, b_bin)

    return out


# speedup vs baseline: 1.2353x; 1.2353x over previous
import jax
import jax.numpy as jnp
from jax.experimental import pallas as pl
from jax.experimental.pallas import tpu as pltpu


_GB = 8
_SB = 64


def _l2norm_rows0(x):
    s = jnp.sum(x * x, axis=0, keepdims=True)
    return x * jax.lax.rsqrt(jnp.maximum(s, 1e-12))


def _tnull_body(w_ref, worlds_ref, t_ref):
    w = w_ref[...]
    x = jax.lax.dot_general(w, worlds_ref[...], (((2,), (1,)), ((), ())),
                            preferred_element_type=jnp.float32)
    s = jnp.sum(x * x, axis=1, keepdims=True)
    t_ref[...] = x * jax.lax.rsqrt(jnp.maximum(s, 1e-12))


def _make_main_body(P, B, D, NW, Gb):
    def body(startsN_ref, startsU_ref, startsB_ref,
             symN_ref, rowN_ref,
             symU_ref, a0U_ref, rowU_ref,
             symB_ref, a0B_ref, a1B_ref, rowB_ref,
             cs_ref, tn_ref, wun_ref, bun_ref, wbin_ref, bbin_ref,
             out_ref):
        j = pl.program_id(0)
        base = j * Gb
        out_ref[...] = jnp.zeros((Gb, D, NW), jnp.float32)

        def clb(lb):
            return jnp.minimum(jnp.maximum(lb, 0), Gb - 1)

        def pair_loop(lo, hi, nmax, compute_y, row_ref):
            def one(i, scale):
                y = compute_y(i) * scale
                lb = clb(row_ref[i] - base)
                out_ref[pl.ds(lb, 1)] = out_ref[pl.ds(lb, 1)] + y[None]

            def pbody(k, carry):
                i = lo + 2 * k
                i2 = jnp.minimum(i + 1, nmax - 1)
                one(i, 1.0)
                one(i2, jnp.where(i + 1 < hi, 1.0, 0.0))
                return carry

            jax.lax.fori_loop(0, (hi - lo + 1) // 2, pbody, 0, unroll=False)

        def null_y(i):
            s = symN_ref[i]
            return tn_ref[pl.ds(s, 1)][0]

        def unary_y(i):
            s = symU_ref[i]
            a = a0U_ref[i]
            lb = clb(rowU_ref[i] - base)
            x = cs_ref[pl.ds(a, 1), pl.ds(lb, 1)][0, 0]
            w = wun_ref[pl.ds(s, 1)][0]
            y = jax.lax.dot_general(w, x, (((1,), (0,)), ((), ())),
                                    preferred_element_type=jnp.float32)
            b = bun_ref[pl.ds(s, 1)][0]
            y = y + jax.lax.broadcast_in_dim(b, (D, NW), (0,))
            return _l2norm_rows0(y)

        def binary_y(i):
            s = symB_ref[i]
            a0 = a0B_ref[i]
            a1 = a1B_ref[i]
            lb = clb(rowB_ref[i] - base)
            xl = cs_ref[pl.ds(a0, 1), pl.ds(lb, 1)][0, 0]
            xr = cs_ref[pl.ds(a1, 1), pl.ds(lb, 1)][0, 0]
            w = wbin_ref[pl.ds(s, 1)][0]
            y = (jax.lax.dot_general(w[:, :D], xl, (((1,), (0,)), ((), ())),
                                     preferred_element_type=jnp.float32)
                 + jax.lax.dot_general(w[:, D:], xr, (((1,), (0,)), ((), ())),
                                       preferred_element_type=jnp.float32))
            b = bbin_ref[pl.ds(s, 1)][0]
            y = y + jax.lax.broadcast_in_dim(b, (D, NW), (0,))
            return _l2norm_rows0(y)

        n0 = symN_ref.shape[0]
        n1 = symU_ref.shape[0]
        n2 = symB_ref.shape[0]
        pair_loop(startsN_ref[j], startsN_ref[j + 1], n0, null_y, rowN_ref)
        pair_loop(startsU_ref[j], startsU_ref[j + 1], n1, unary_y, rowU_ref)
        pair_loop(startsB_ref[j], startsB_ref[j + 1], n2, binary_y, rowB_ref)

    return body


def kernel(worlds, computed_states, null_indices, null_symbols,
           unary_indices, unary_symbols, unary_args,
           binary_indices, binary_symbols, binary_args,
           W_null, W_un, b_un, W_bin, b_bin):
    P, B, D, NW = computed_states.shape
    S = W_null.shape[0]
    i32 = jnp.int32
    Gb = _GB
    nblk = B // Gb

    T_null = pl.pallas_call(
        _tnull_body,
        grid=(S // _SB,),
        in_specs=[
            pl.BlockSpec((_SB, D, D), lambda i: (i, 0, 0)),
            pl.BlockSpec((NW, D), lambda i: (0, 0)),
        ],
        out_specs=pl.BlockSpec((_SB, D, NW), lambda i: (i, 0, 0)),
        out_shape=jax.ShapeDtypeStruct((S, D, NW), jnp.float32),
        name="sat3_tnull",
    )(W_null, worlds)

    def prep_packed(idx, payloads, widths):
        key = idx.astype(i32)
        for p, w in zip(payloads, widths):
            key = (key << w) | p.astype(i32)
        key = jnp.sort(key)
        tot = sum(widths)
        row = key >> tot
        starts = jnp.searchsorted(
            row, jnp.arange(0, B + 1, Gb, dtype=i32)).astype(i32)
        outs = []
        rem = key
        for w in reversed(widths):
            outs.append(rem & ((1 << w) - 1))
            rem = rem >> w
        return (starts, row) + tuple(reversed(outs))

    sym_bits = max(1, (S - 1).bit_length())
    arg_bits = max(1, (P - 1).bit_length())
    startsN, rowN, symN = prep_packed(
        null_indices, [null_symbols], [sym_bits])
    startsU, rowU, symU, a0U = prep_packed(
        unary_indices, [unary_symbols, unary_args], [sym_bits, arg_bits])
    startsB, rowB, symB, a0B, a1B = prep_packed(
        binary_indices, [binary_symbols, binary_args[:, 0], binary_args[:, 1]],
        [sym_bits, arg_bits, arg_bits])

    grid_spec = pltpu.PrefetchScalarGridSpec(
        num_scalar_prefetch=12,
        grid=(nblk,),
        in_specs=[
            pl.BlockSpec((P, Gb, D, NW),
                         lambda j, *_: (0, j, 0, 0)),
            pl.BlockSpec(memory_space=pltpu.MemorySpace.VMEM),
            pl.BlockSpec(memory_space=pltpu.MemorySpace.VMEM),
            pl.BlockSpec(memory_space=pltpu.MemorySpace.VMEM),
            pl.BlockSpec(memory_space=pltpu.MemorySpace.VMEM),
            pl.BlockSpec(memory_space=pltpu.MemorySpace.VMEM),
        ],
        out_specs=pl.BlockSpec((Gb, D, NW), lambda j, *_: (j, 0, 0)),
    )

    out = pl.pallas_call(
        _make_main_body(P, B, D, NW, Gb),
        grid_spec=grid_spec,
        out_shape=jax.ShapeDtypeStruct((B, D, NW), jnp.float32),
        name="sat3_main",
    )(startsN, startsU, startsB, symN, rowN, symU, a0U, rowU,
      symB, a0B, a1B, rowB,
      computed_states, T_null, W_un, b_un, W_bin, b_bin)

    return out


# speedup vs baseline: 1.3148x; 1.0644x over previous
import jax
import jax.numpy as jnp
from jax.experimental import pallas as pl
from jax.experimental.pallas import tpu as pltpu


_GB = 8
_SB = 64


def _l2norm_rows0(x):
    s = jnp.sum(x * x, axis=0, keepdims=True)
    return x * jax.lax.rsqrt(jnp.maximum(s, 1e-12))


def _tnull_body(w_ref, worlds_ref, t_ref):
    w = w_ref[...]
    x = jax.lax.dot_general(w, worlds_ref[...], (((2,), (1,)), ((), ())),
                            preferred_element_type=jnp.float32)
    s = jnp.sum(x * x, axis=1, keepdims=True)
    t_ref[...] = x * jax.lax.rsqrt(jnp.maximum(s, 1e-12))


def _make_main_body(P, B, D, NW, Gb):
    def body(startsN_ref, startsU_ref, startsB_ref,
             symN_ref, rowN_ref,
             symU_ref, a0U_ref, rowU_ref,
             symB_ref, a0B_ref, a1B_ref, rowB_ref,
             cs_ref, tn_ref, wun_ref, bun_ref, wbin_ref, bbin_ref,
             out_ref):
        j = pl.program_id(0)
        base = j * Gb
        out_ref[...] = jnp.zeros((Gb, D, NW), jnp.float32)

        def clb(lb):
            return jnp.minimum(jnp.maximum(lb, 0), Gb - 1)

        def pair_loop(lo, hi, nmax, compute_y, row_ref):
            def one(i, scale):
                y = compute_y(i) * scale
                lb = clb(row_ref[i] - base)
                out_ref[pl.ds(lb, 1)] = out_ref[pl.ds(lb, 1)] + y[None]

            def pbody(k, carry):
                i = lo + 4 * k
                one(i, 1.0)
                for d in range(1, 4):
                    one(jnp.minimum(i + d, nmax - 1),
                        jnp.where(i + d < hi, 1.0, 0.0))
                return carry

            jax.lax.fori_loop(0, (hi - lo + 3) // 4, pbody, 0, unroll=False)

        def null_y(i):
            s = symN_ref[i]
            return tn_ref[pl.ds(s, 1)][0]

        def unary_y(i):
            s = symU_ref[i]
            a = a0U_ref[i]
            lb = clb(rowU_ref[i] - base)
            x = cs_ref[pl.ds(a, 1), pl.ds(lb, 1)][0, 0]
            w = wun_ref[pl.ds(s, 1)][0]
            y = jax.lax.dot_general(w, x, (((1,), (0,)), ((), ())),
                                    preferred_element_type=jnp.float32)
            b = bun_ref[pl.ds(s, 1)][0]
            y = y + jax.lax.broadcast_in_dim(b, (D, NW), (0,))
            return _l2norm_rows0(y)

        def binary_y(i):
            s = symB_ref[i]
            a0 = a0B_ref[i]
            a1 = a1B_ref[i]
            lb = clb(rowB_ref[i] - base)
            xl = cs_ref[pl.ds(a0, 1), pl.ds(lb, 1)][0, 0]
            xr = cs_ref[pl.ds(a1, 1), pl.ds(lb, 1)][0, 0]
            w = wbin_ref[pl.ds(s, 1)][0]
            y = (jax.lax.dot_general(w[:, :D], xl, (((1,), (0,)), ((), ())),
                                     preferred_element_type=jnp.float32)
                 + jax.lax.dot_general(w[:, D:], xr, (((1,), (0,)), ((), ())),
                                       preferred_element_type=jnp.float32))
            b = bbin_ref[pl.ds(s, 1)][0]
            y = y + jax.lax.broadcast_in_dim(b, (D, NW), (0,))
            return _l2norm_rows0(y)

        n0 = symN_ref.shape[0]
        n1 = symU_ref.shape[0]
        n2 = symB_ref.shape[0]
        pair_loop(startsN_ref[j], startsN_ref[j + 1], n0, null_y, rowN_ref)
        pair_loop(startsU_ref[j], startsU_ref[j + 1], n1, unary_y, rowU_ref)
        pair_loop(startsB_ref[j], startsB_ref[j + 1], n2, binary_y, rowB_ref)

    return body


def kernel(worlds, computed_states, null_indices, null_symbols,
           unary_indices, unary_symbols, unary_args,
           binary_indices, binary_symbols, binary_args,
           W_null, W_un, b_un, W_bin, b_bin):
    P, B, D, NW = computed_states.shape
    S = W_null.shape[0]
    i32 = jnp.int32
    Gb = _GB
    nblk = B // Gb

    T_null = pl.pallas_call(
        _tnull_body,
        grid=(S // _SB,),
        in_specs=[
            pl.BlockSpec((_SB, D, D), lambda i: (i, 0, 0)),
            pl.BlockSpec((NW, D), lambda i: (0, 0)),
        ],
        out_specs=pl.BlockSpec((_SB, D, NW), lambda i: (i, 0, 0)),
        out_shape=jax.ShapeDtypeStruct((S, D, NW), jnp.float32),
        name="sat3_tnull",
    )(W_null, worlds)

    def prep_packed(idx, payloads, widths):
        key = idx.astype(i32)
        for p, w in zip(payloads, widths):
            key = (key << w) | p.astype(i32)
        key = jnp.sort(key)
        tot = sum(widths)
        row = key >> tot
        starts = jnp.searchsorted(
            row, jnp.arange(0, B + 1, Gb, dtype=i32)).astype(i32)
        outs = []
        rem = key
        for w in reversed(widths):
            outs.append(rem & ((1 << w) - 1))
            rem = rem >> w
        return (starts, row) + tuple(reversed(outs))

    sym_bits = max(1, (S - 1).bit_length())
    arg_bits = max(1, (P - 1).bit_length())
    startsN, rowN, symN = prep_packed(
        null_indices, [null_symbols], [sym_bits])
    startsU, rowU, symU, a0U = prep_packed(
        unary_indices, [unary_symbols, unary_args], [sym_bits, arg_bits])
    startsB, rowB, symB, a0B, a1B = prep_packed(
        binary_indices, [binary_symbols, binary_args[:, 0], binary_args[:, 1]],
        [sym_bits, arg_bits, arg_bits])

    grid_spec = pltpu.PrefetchScalarGridSpec(
        num_scalar_prefetch=12,
        grid=(nblk,),
        in_specs=[
            pl.BlockSpec((P, Gb, D, NW),
                         lambda j, *_: (0, j, 0, 0)),
            pl.BlockSpec(memory_space=pltpu.MemorySpace.VMEM),
            pl.BlockSpec(memory_space=pltpu.MemorySpace.VMEM),
            pl.BlockSpec(memory_space=pltpu.MemorySpace.VMEM),
            pl.BlockSpec(memory_space=pltpu.MemorySpace.VMEM),
            pl.BlockSpec(memory_space=pltpu.MemorySpace.VMEM),
        ],
        out_specs=pl.BlockSpec((Gb, D, NW), lambda j, *_: (j, 0, 0)),
    )

    out = pl.pallas_call(
        _make_main_body(P, B, D, NW, Gb),
        grid_spec=grid_spec,
        out_shape=jax.ShapeDtypeStruct((B, D, NW), jnp.float32),
        name="sat3_main",
    )(startsN, startsU, startsB, symN, rowN, symU, a0U, rowU,
      symB, a0B, a1B, rowB,
      computed_states, T_null, W_un, b_un, W_bin, b_bin)

    return out


# speedup vs baseline: 1.8317x; 1.3931x over previous
import jax
import jax.numpy as jnp
from jax.experimental import pallas as pl
from jax.experimental.pallas import tpu as pltpu


_GB = 32
_SB = 64


def _l2norm_rows0(x):
    s = jnp.sum(x * x, axis=0, keepdims=True)
    return x * jax.lax.rsqrt(jnp.maximum(s, 1e-12))


def _tnull_body(w_ref, worlds_ref, t_ref):
    w = w_ref[...]
    x = jax.lax.dot_general(w, worlds_ref[...], (((2,), (1,)), ((), ())),
                            preferred_element_type=jnp.float32)
    s = jnp.sum(x * x, axis=1, keepdims=True)
    t_ref[...] = x * jax.lax.rsqrt(jnp.maximum(s, 1e-12))


def _make_main_body(P, B, D, NW, Gb):
    def body(startsN_ref, startsU_ref, startsB_ref,
             symN_ref, rowN_ref,
             symU_ref, a0U_ref, rowU_ref,
             symB_ref, a0B_ref, a1B_ref, rowB_ref,
             cs_ref, tn_ref, wun_ref, bun_ref, wbin_ref, bbin_ref,
             out_ref):
        j = pl.program_id(0)
        base = j * Gb
        out_ref[...] = jnp.zeros((Gb, D, NW), jnp.float32)

        def clb(lb):
            return jnp.minimum(jnp.maximum(lb, 0), Gb - 1)

        def pair_loop(lo, hi, nmax, compute_y, row_ref):
            def one(i, scale):
                y = compute_y(i) * scale
                lb = clb(row_ref[i] - base)
                out_ref[pl.ds(lb, 1)] = out_ref[pl.ds(lb, 1)] + y[None]

            def pbody(k, carry):
                i = lo + 4 * k
                one(i, 1.0)
                for d in range(1, 4):
                    one(jnp.minimum(i + d, nmax - 1),
                        jnp.where(i + d < hi, 1.0, 0.0))
                return carry

            jax.lax.fori_loop(0, (hi - lo + 3) // 4, pbody, 0, unroll=False)

        def null_y(i):
            s = symN_ref[i]
            return tn_ref[pl.ds(s, 1)][0]

        def unary_y(i):
            s = symU_ref[i]
            a = a0U_ref[i]
            lb = clb(rowU_ref[i] - base)
            x = cs_ref[pl.ds(a, 1), pl.ds(lb, 1)][0, 0]
            w = wun_ref[pl.ds(s, 1)][0]
            y = jax.lax.dot_general(w, x, (((1,), (0,)), ((), ())),
                                    preferred_element_type=jnp.float32)
            b = bun_ref[pl.ds(s, 1)][0]
            y = y + jax.lax.broadcast_in_dim(b, (D, NW), (0,))
            return _l2norm_rows0(y)

        def binary_y(i):
            s = symB_ref[i]
            a0 = a0B_ref[i]
            a1 = a1B_ref[i]
            lb = clb(rowB_ref[i] - base)
            xl = cs_ref[pl.ds(a0, 1), pl.ds(lb, 1)][0, 0]
            xr = cs_ref[pl.ds(a1, 1), pl.ds(lb, 1)][0, 0]
            w = wbin_ref[pl.ds(s, 1)][0]
            y = (jax.lax.dot_general(w[:, :D], xl, (((1,), (0,)), ((), ())),
                                     preferred_element_type=jnp.float32)
                 + jax.lax.dot_general(w[:, D:], xr, (((1,), (0,)), ((), ())),
                                       preferred_element_type=jnp.float32))
            b = bbin_ref[pl.ds(s, 1)][0]
            y = y + jax.lax.broadcast_in_dim(b, (D, NW), (0,))
            return _l2norm_rows0(y)

        n0 = symN_ref.shape[0]
        n1 = symU_ref.shape[0]
        n2 = symB_ref.shape[0]
        pair_loop(startsN_ref[j], startsN_ref[j + 1], n0, null_y, rowN_ref)
        pair_loop(startsU_ref[j], startsU_ref[j + 1], n1, unary_y, rowU_ref)
        pair_loop(startsB_ref[j], startsB_ref[j + 1], n2, binary_y, rowB_ref)

    return body


def kernel(worlds, computed_states, null_indices, null_symbols,
           unary_indices, unary_symbols, unary_args,
           binary_indices, binary_symbols, binary_args,
           W_null, W_un, b_un, W_bin, b_bin):
    P, B, D, NW = computed_states.shape
    S = W_null.shape[0]
    i32 = jnp.int32
    Gb = _GB
    nblk = B // Gb

    T_null = pl.pallas_call(
        _tnull_body,
        grid=(S // _SB,),
        in_specs=[
            pl.BlockSpec((_SB, D, D), lambda i: (i, 0, 0)),
            pl.BlockSpec((NW, D), lambda i: (0, 0)),
        ],
        out_specs=pl.BlockSpec((_SB, D, NW), lambda i: (i, 0, 0)),
        out_shape=jax.ShapeDtypeStruct((S, D, NW), jnp.float32),
        name="sat3_tnull",
    )(W_null, worlds)

    def prep_packed(idx, payloads, widths):
        key = idx.astype(i32)
        for p, w in zip(payloads, widths):
            key = (key << w) | p.astype(i32)
        key = jnp.sort(key)
        tot = sum(widths)
        row = key >> tot
        starts = jnp.searchsorted(
            row, jnp.arange(0, B + 1, Gb, dtype=i32)).astype(i32)
        outs = []
        rem = key
        for w in reversed(widths):
            outs.append(rem & ((1 << w) - 1))
            rem = rem >> w
        return (starts, row) + tuple(reversed(outs))

    sym_bits = max(1, (S - 1).bit_length())
    arg_bits = max(1, (P - 1).bit_length())
    startsN, rowN, symN = prep_packed(
        null_indices, [null_symbols], [sym_bits])
    startsU, rowU, symU, a0U = prep_packed(
        unary_indices, [unary_symbols, unary_args], [sym_bits, arg_bits])
    startsB, rowB, symB, a0B, a1B = prep_packed(
        binary_indices, [binary_symbols, binary_args[:, 0], binary_args[:, 1]],
        [sym_bits, arg_bits, arg_bits])

    grid_spec = pltpu.PrefetchScalarGridSpec(
        num_scalar_prefetch=12,
        grid=(nblk,),
        in_specs=[
            pl.BlockSpec((P, Gb, D, NW),
                         lambda j, *_: (0, j, 0, 0)),
            pl.BlockSpec(memory_space=pltpu.MemorySpace.VMEM),
            pl.BlockSpec(memory_space=pltpu.MemorySpace.VMEM),
            pl.BlockSpec(memory_space=pltpu.MemorySpace.VMEM),
            pl.BlockSpec(memory_space=pltpu.MemorySpace.VMEM),
            pl.BlockSpec(memory_space=pltpu.MemorySpace.VMEM),
        ],
        out_specs=pl.BlockSpec((Gb, D, NW), lambda j, *_: (j, 0, 0)),
    )

    out = pl.pallas_call(
        _make_main_body(P, B, D, NW, Gb),
        grid_spec=grid_spec,
        out_shape=jax.ShapeDtypeStruct((B, D, NW), jnp.float32),
        name="sat3_main",
    )(startsN, startsU, startsB, symN, rowN, symU, a0U, rowU,
      symB, a0B, a1B, rowB,
      computed_states, T_null, W_un, b_un, W_bin, b_bin)

    return out


# speedup vs baseline: 1.9510x; 1.0651x over previous
import jax
import jax.numpy as jnp
from jax.experimental import pallas as pl
from jax.experimental.pallas import tpu as pltpu


_GB = 32
_SB = 64


def _l2norm_rows0(x):
    s = jnp.sum(x * x, axis=0, keepdims=True)
    return x * jax.lax.rsqrt(jnp.maximum(s, 1e-12))


def _tnull_body(w_ref, worlds_ref, t_ref):
    w = w_ref[...]
    x = jax.lax.dot_general(w, worlds_ref[...], (((2,), (1,)), ((), ())),
                            preferred_element_type=jnp.float32)
    s = jnp.sum(x * x, axis=1, keepdims=True)
    t_ref[...] = x * jax.lax.rsqrt(jnp.maximum(s, 1e-12))


def _make_main_body(P, B, D, NW, Gb):
    def body(startsN_ref, startsU_ref, startsB_ref,
             symN_ref, rowN_ref,
             symU_ref, a0U_ref, rowU_ref,
             symB_ref, a0B_ref, a1B_ref, rowB_ref,
             cs_ref, tn_ref, wun_ref, bun_ref, wbin_ref, bbin_ref,
             out_ref):
        j = pl.program_id(0)
        base = j * Gb
        out_ref[...] = jnp.zeros((Gb, D, NW), jnp.float32)

        def clb(lb):
            return jnp.minimum(jnp.maximum(lb, 0), Gb - 1)

        def pair_loop(lo, hi, nmax, compute_y, row_ref):
            def one(i, scale):
                y = compute_y(i) * scale
                lb = clb(row_ref[i] - base)
                out_ref[pl.ds(lb, 1)] = out_ref[pl.ds(lb, 1)] + y[None]

            def pbody(k, carry):
                i = lo + 4 * k
                one(i, 1.0)
                for d in range(1, 4):
                    one(jnp.minimum(i + d, nmax - 1),
                        jnp.where(i + d < hi, 1.0, 0.0))
                return carry

            jax.lax.fori_loop(0, (hi - lo + 3) // 4, pbody, 0, unroll=False)

        def null_y(i):
            s = symN_ref[i]
            return tn_ref[pl.ds(s, 1)][0]

        def unary_y(i):
            s = symU_ref[i]
            a = a0U_ref[i]
            lb = clb(rowU_ref[i] - base)
            x = cs_ref[pl.ds(a, 1), pl.ds(lb, 1)][0, 0]
            w = wun_ref[pl.ds(s, 1)][0]
            y = jax.lax.dot_general(w, x, (((1,), (0,)), ((), ())),
                                    preferred_element_type=jnp.float32)
            b = bun_ref[pl.ds(s, 1)][0]
            y = y + jax.lax.broadcast_in_dim(b, (D, NW), (0,))
            return _l2norm_rows0(y)

        def binary_y(i):
            s = symB_ref[i]
            a0 = a0B_ref[i]
            a1 = a1B_ref[i]
            lb = clb(rowB_ref[i] - base)
            xl = cs_ref[pl.ds(a0, 1), pl.ds(lb, 1)][0, 0]
            xr = cs_ref[pl.ds(a1, 1), pl.ds(lb, 1)][0, 0]
            w = wbin_ref[pl.ds(s, 1)][0]
            y = (jax.lax.dot_general(w[:, :D], xl, (((1,), (0,)), ((), ())),
                                     preferred_element_type=jnp.float32)
                 + jax.lax.dot_general(w[:, D:], xr, (((1,), (0,)), ((), ())),
                                       preferred_element_type=jnp.float32))
            b = bbin_ref[pl.ds(s, 1)][0]
            y = y + jax.lax.broadcast_in_dim(b, (D, NW), (0,))
            return _l2norm_rows0(y)

        n0 = symN_ref.shape[0]
        n1 = symU_ref.shape[0]
        n2 = symB_ref.shape[0]
        pair_loop(startsN_ref[j], startsN_ref[j + 1], n0, null_y, rowN_ref)
        pair_loop(startsU_ref[j], startsU_ref[j + 1], n1, unary_y, rowU_ref)
        pair_loop(startsB_ref[j], startsB_ref[j + 1], n2, binary_y, rowB_ref)

    return body


def kernel(worlds, computed_states, null_indices, null_symbols,
           unary_indices, unary_symbols, unary_args,
           binary_indices, binary_symbols, binary_args,
           W_null, W_un, b_un, W_bin, b_bin):
    P, B, D, NW = computed_states.shape
    S = W_null.shape[0]
    i32 = jnp.int32
    Gb = _GB
    nblk = B // Gb

    T_null = pl.pallas_call(
        _tnull_body,
        grid=(S // _SB,),
        in_specs=[
            pl.BlockSpec((_SB, D, D), lambda i: (i, 0, 0)),
            pl.BlockSpec((NW, D), lambda i: (0, 0)),
        ],
        out_specs=pl.BlockSpec((_SB, D, NW), lambda i: (i, 0, 0)),
        out_shape=jax.ShapeDtypeStruct((S, D, NW), jnp.float32),
        name="sat3_tnull",
    )(W_null, worlds)

    def prep_packed(idx, payloads, widths):
        key = idx.astype(i32)
        for p, w in zip(payloads, widths):
            key = (key << w) | p.astype(i32)
        key = jnp.sort(key)
        tot = sum(widths)
        row = key >> tot
        bounds = jnp.arange(0, B + 1, Gb, dtype=i32)
        starts = jnp.sum(row[None, :] < bounds[:, None], axis=1,
                         dtype=i32)
        outs = []
        rem = key
        for w in reversed(widths):
            outs.append(rem & ((1 << w) - 1))
            rem = rem >> w
        return (starts, row) + tuple(reversed(outs))

    sym_bits = max(1, (S - 1).bit_length())
    arg_bits = max(1, (P - 1).bit_length())
    startsN, rowN, symN = prep_packed(
        null_indices, [null_symbols], [sym_bits])
    startsU, rowU, symU, a0U = prep_packed(
        unary_indices, [unary_symbols, unary_args], [sym_bits, arg_bits])
    startsB, rowB, symB, a0B, a1B = prep_packed(
        binary_indices, [binary_symbols, binary_args[:, 0], binary_args[:, 1]],
        [sym_bits, arg_bits, arg_bits])

    grid_spec = pltpu.PrefetchScalarGridSpec(
        num_scalar_prefetch=12,
        grid=(nblk,),
        in_specs=[
            pl.BlockSpec((P, Gb, D, NW),
                         lambda j, *_: (0, j, 0, 0)),
            pl.BlockSpec(memory_space=pltpu.MemorySpace.VMEM),
            pl.BlockSpec(memory_space=pltpu.MemorySpace.VMEM),
            pl.BlockSpec(memory_space=pltpu.MemorySpace.VMEM),
            pl.BlockSpec(memory_space=pltpu.MemorySpace.VMEM),
            pl.BlockSpec(memory_space=pltpu.MemorySpace.VMEM),
        ],
        out_specs=pl.BlockSpec((Gb, D, NW), lambda j, *_: (j, 0, 0)),
    )

    out = pl.pallas_call(
        _make_main_body(P, B, D, NW, Gb),
        grid_spec=grid_spec,
        out_shape=jax.ShapeDtypeStruct((B, D, NW), jnp.float32),
        name="sat3_main",
    )(startsN, startsU, startsB, symN, rowN, symU, a0U, rowU,
      symB, a0B, a1B, rowB,
      computed_states, T_null, W_un, b_un, W_bin, b_bin)

    return out


# speedup vs baseline: 2.0653x; 1.0586x over previous
import jax
import jax.numpy as jnp
from jax.experimental import pallas as pl
from jax.experimental.pallas import tpu as pltpu


_GB = 32
_SB = 64


def _l2norm_rows0(x):
    s = jnp.sum(x * x, axis=0, keepdims=True)
    return x * jax.lax.rsqrt(jnp.maximum(s, 1e-12))


def _tnull_body(w_ref, worlds_ref, t_ref):
    w = w_ref[...]
    x = jax.lax.dot_general(w, worlds_ref[...], (((2,), (1,)), ((), ())),
                            preferred_element_type=jnp.float32)
    s = jnp.sum(x * x, axis=1, keepdims=True)
    t_ref[...] = x * jax.lax.rsqrt(jnp.maximum(s, 1e-12))


def _make_main_body(P, B, D, NW, Gb):
    def body(startsN_ref, startsU_ref, startsB_ref,
             symN_ref, rowN_ref,
             symU_ref, a0U_ref, rowU_ref,
             symB_ref, a0B_ref, a1B_ref, rowB_ref,
             cs_ref, tn_ref, wun_ref, bun_ref, wbin_ref, bbin_ref,
             out_ref):
        j = pl.program_id(0)
        base = j * Gb
        out_ref[...] = jnp.zeros((Gb, D, NW), jnp.float32)

        def clb(lb):
            return jnp.minimum(jnp.maximum(lb, 0), Gb - 1)

        def pair_loop(lo, hi, nmax, compute_y, row_ref):
            def one(i, scale):
                y = compute_y(i) * scale
                lb = clb(row_ref[i] - base)
                out_ref[pl.ds(lb, 1)] = out_ref[pl.ds(lb, 1)] + y[None]

            def pbody(k, carry):
                i = lo + 8 * k
                one(i, 1.0)
                for d in range(1, 8):
                    one(jnp.minimum(i + d, nmax - 1),
                        jnp.where(i + d < hi, 1.0, 0.0))
                return carry

            jax.lax.fori_loop(0, (hi - lo + 7) // 8, pbody, 0, unroll=False)

        def null_y(i):
            s = symN_ref[i]
            return tn_ref[pl.ds(s, 1)][0]

        def unary_y(i):
            s = symU_ref[i]
            a = a0U_ref[i]
            lb = clb(rowU_ref[i] - base)
            x = cs_ref[pl.ds(a, 1), pl.ds(lb, 1)][0, 0]
            w = wun_ref[pl.ds(s, 1)][0]
            y = jax.lax.dot_general(w, x, (((1,), (0,)), ((), ())),
                                    preferred_element_type=jnp.float32)
            b = bun_ref[pl.ds(s, 1)][0]
            y = y + jax.lax.broadcast_in_dim(b, (D, NW), (0,))
            return _l2norm_rows0(y)

        def binary_y(i):
            s = symB_ref[i]
            a0 = a0B_ref[i]
            a1 = a1B_ref[i]
            lb = clb(rowB_ref[i] - base)
            xl = cs_ref[pl.ds(a0, 1), pl.ds(lb, 1)][0, 0]
            xr = cs_ref[pl.ds(a1, 1), pl.ds(lb, 1)][0, 0]
            w = wbin_ref[pl.ds(s, 1)][0]
            y = (jax.lax.dot_general(w[:, :D], xl, (((1,), (0,)), ((), ())),
                                     preferred_element_type=jnp.float32)
                 + jax.lax.dot_general(w[:, D:], xr, (((1,), (0,)), ((), ())),
                                       preferred_element_type=jnp.float32))
            b = bbin_ref[pl.ds(s, 1)][0]
            y = y + jax.lax.broadcast_in_dim(b, (D, NW), (0,))
            return _l2norm_rows0(y)

        n0 = symN_ref.shape[0]
        n1 = symU_ref.shape[0]
        n2 = symB_ref.shape[0]
        pair_loop(startsN_ref[j], startsN_ref[j + 1], n0, null_y, rowN_ref)
        pair_loop(startsU_ref[j], startsU_ref[j + 1], n1, unary_y, rowU_ref)
        pair_loop(startsB_ref[j], startsB_ref[j + 1], n2, binary_y, rowB_ref)

    return body


def kernel(worlds, computed_states, null_indices, null_symbols,
           unary_indices, unary_symbols, unary_args,
           binary_indices, binary_symbols, binary_args,
           W_null, W_un, b_un, W_bin, b_bin):
    P, B, D, NW = computed_states.shape
    S = W_null.shape[0]
    i32 = jnp.int32
    Gb = _GB
    nblk = B // Gb

    T_null = pl.pallas_call(
        _tnull_body,
        grid=(S // _SB,),
        in_specs=[
            pl.BlockSpec((_SB, D, D), lambda i: (i, 0, 0)),
            pl.BlockSpec((NW, D), lambda i: (0, 0)),
        ],
        out_specs=pl.BlockSpec((_SB, D, NW), lambda i: (i, 0, 0)),
        out_shape=jax.ShapeDtypeStruct((S, D, NW), jnp.float32),
        name="sat3_tnull",
    )(W_null, worlds)

    def prep_packed(idx, payloads, widths):
        key = idx.astype(i32)
        for p, w in zip(payloads, widths):
            key = (key << w) | p.astype(i32)
        key = jnp.sort(key)
        tot = sum(widths)
        row = key >> tot
        bounds = jnp.arange(0, B + 1, Gb, dtype=i32)
        starts = jnp.sum(row[None, :] < bounds[:, None], axis=1,
                         dtype=i32)
        outs = []
        rem = key
        for w in reversed(widths):
            outs.append(rem & ((1 << w) - 1))
            rem = rem >> w
        return (starts, row) + tuple(reversed(outs))

    sym_bits = max(1, (S - 1).bit_length())
    arg_bits = max(1, (P - 1).bit_length())
    startsN, rowN, symN = prep_packed(
        null_indices, [null_symbols], [sym_bits])
    startsU, rowU, symU, a0U = prep_packed(
        unary_indices, [unary_symbols, unary_args], [sym_bits, arg_bits])
    startsB, rowB, symB, a0B, a1B = prep_packed(
        binary_indices, [binary_symbols, binary_args[:, 0], binary_args[:, 1]],
        [sym_bits, arg_bits, arg_bits])

    grid_spec = pltpu.PrefetchScalarGridSpec(
        num_scalar_prefetch=12,
        grid=(nblk,),
        in_specs=[
            pl.BlockSpec((P, Gb, D, NW),
                         lambda j, *_: (0, j, 0, 0)),
            pl.BlockSpec(memory_space=pltpu.MemorySpace.VMEM),
            pl.BlockSpec(memory_space=pltpu.MemorySpace.VMEM),
            pl.BlockSpec(memory_space=pltpu.MemorySpace.VMEM),
            pl.BlockSpec(memory_space=pltpu.MemorySpace.VMEM),
            pl.BlockSpec(memory_space=pltpu.MemorySpace.VMEM),
        ],
        out_specs=pl.BlockSpec((Gb, D, NW), lambda j, *_: (j, 0, 0)),
    )

    out = pl.pallas_call(
        _make_main_body(P, B, D, NW, Gb),
        grid_spec=grid_spec,
        out_shape=jax.ShapeDtypeStruct((B, D, NW), jnp.float32),
        name="sat3_main",
    )(startsN, startsU, startsB, symN, rowN, symU, a0U, rowU,
      symB, a0B, a1B, rowB,
      computed_states, T_null, W_un, b_un, W_bin, b_bin)

    return out
